# baseline (speedup 1.0000x reference)
import functools

import jax
import jax.numpy as jnp
from jax.experimental import pallas as pl
from jax.experimental.pallas import tpu as pltpu

_BM = 256


def _matmul_block(h_ref, wq_ref, out_ref):
    out_ref[...] = jax.lax.dot_general(
        h_ref[...],
        wq_ref[...],
        dimension_numbers=(((1,), (1,)), ((), ())),
        preferred_element_type=jnp.float32,
        precision=jax.lax.Precision.DEFAULT,
    )


@jax.jit
def kernel(h, Wq, Wn):
    del Wn
    m, d = h.shape
    e = Wq.shape[0]
    grid = (m // _BM,)
    return pl.pallas_call(
        _matmul_block,
        grid=grid,
        in_specs=[
            pl.BlockSpec((_BM, d), lambda i: (i, 0)),
            pl.BlockSpec((e, d), lambda i: (0, 0)),
        ],
        out_specs=pl.BlockSpec((_BM, e), lambda i: (i, 0)),
        out_shape=jax.ShapeDtypeStruct((m, e), jnp.float32),
        compiler_params=pltpu.CompilerParams(
            dimension_semantics=("arbitrary",),
        ),
    )(h, Wq)


# speedup vs baseline: 1.1960x; 1.1960x over previous
import jax
import jax.numpy as jnp
from jax.experimental import pallas as pl
from jax.experimental.pallas import tpu as pltpu

_BM = 512


def _matmul_block(ha_ref, hb_ref, wq_ref, out_ref):
    dn = (((1,), (1,)), ((), ()))
    out_ref[:_BM, :] = jax.lax.dot_general(
        ha_ref[...], wq_ref[...], dimension_numbers=dn,
        preferred_element_type=jnp.float32,
        precision=jax.lax.Precision.DEFAULT,
    )
    out_ref[_BM:, :] = jax.lax.dot_general(
        hb_ref[...], wq_ref[...], dimension_numbers=dn,
        preferred_element_type=jnp.float32,
        precision=jax.lax.Precision.DEFAULT,
    )


@jax.jit
def kernel(h, Wq, Wn):
    del Wn
    m, d = h.shape
    e = Wq.shape[0]
    grid = (m // (2 * _BM),)
    return pl.pallas_call(
        _matmul_block,
        grid=grid,
        in_specs=[
            pl.BlockSpec((_BM, d), lambda i: (2 * i, 0)),
            pl.BlockSpec((_BM, d), lambda i: (2 * i + 1, 0)),
            pl.BlockSpec((e, d), lambda i: (0, 0)),
        ],
        out_specs=pl.BlockSpec((2 * _BM, e), lambda i: (i, 0)),
        out_shape=jax.ShapeDtypeStruct((m, e), jnp.float32),
        compiler_params=pltpu.CompilerParams(
            dimension_semantics=("arbitrary",),
        ),
    )(h, h, Wq)


# speedup vs baseline: 1.2038x; 1.0066x over previous
import jax
import jax.numpy as jnp
from jax.experimental import pallas as pl
from jax.experimental.pallas import tpu as pltpu

_BM = 512


def _matmul_block(h_ref, wq_ref, out_ref):
    out_ref[...] = jax.lax.dot_general(
        h_ref[...],
        wq_ref[...],
        dimension_numbers=(((1,), (1,)), ((), ())),
        preferred_element_type=jnp.float32,
        precision=jax.lax.Precision.DEFAULT,
    )


@jax.jit
def kernel(h, Wq, Wn):
    del Wn
    m, d = h.shape
    e = Wq.shape[0]
    grid = (m // _BM,)
    return pl.pallas_call(
        _matmul_block,
        grid=grid,
        in_specs=[
            pl.BlockSpec((_BM, d), lambda i: (i, 0),
                         ),
            pl.BlockSpec((e, d), lambda i: (0, 0)),
        ],
        out_specs=pl.BlockSpec((_BM, e), lambda i: (i, 0)),
        out_shape=jax.ShapeDtypeStruct((m, e), jnp.float32),
        compiler_params=pltpu.CompilerParams(
            dimension_semantics=("parallel",),
        ),
    )(h, Wq)
